# baseline (speedup 1.0000x reference)
import functools

import jax
import jax.numpy as jnp
from jax import lax
from jax.experimental import pallas as pl
from jax.experimental.pallas import tpu as pltpu
from jax.experimental.pallas import tpu_sc as plsc

_NB = 4
_LA = 2


def _make_sc_kernel(R_sc, T, D):
    info = plsc.get_sparse_core_info()
    NC, NS, L = info.num_cores, info.num_subcores, info.num_lanes
    NW = NC * NS
    rows_per_w = R_sc // NW
    CR = 8
    n_chunks = rows_per_w // CR
    n_col = D // L

    mesh = plsc.VectorSubcoreMesh(core_axis_name="c", subcore_axis_name="s")

    scratch = (
        [pltpu.VMEM((CR, D), jnp.float32) for _ in range(_NB)]
        + [pltpu.VMEM((CR, D), jnp.float32) for _ in range(_NB)]
        + [pltpu.SemaphoreType.DMA for _ in range(3 * _NB)]
    )

    @functools.partial(
        pl.kernel,
        mesh=mesh,
        out_type=jax.ShapeDtypeStruct((R_sc, D), jnp.float32),
        scratch_types=scratch,
    )
    def k(x_hbm, pe_hbm, out_hbm, *refs):
        bufx = refs[:_NB]
        bufp = refs[_NB:2 * _NB]
        sem_lx = refs[2 * _NB:2 * _NB + _NB]
        sem_lp = refs[2 * _NB + _NB:2 * _NB + 2 * _NB]
        sem_st = refs[2 * _NB + 2 * _NB:]

        wid = lax.axis_index("s") * NC + lax.axis_index("c")
        base = wid * rows_per_w

        def start_loads(i, b):
            r0 = base + i * CR
            t0 = lax.rem(r0, T)
            pltpu.async_copy(x_hbm.at[pl.ds(r0, CR)], bufx[b], sem_lx[b])
            pltpu.async_copy(pe_hbm.at[pl.ds(t0, CR)], bufp[b], sem_lp[b])

        def wait_loads(i, b):
            r0 = base + i * CR
            t0 = lax.rem(r0, T)
            pltpu.make_async_copy(x_hbm.at[pl.ds(r0, CR)], bufx[b], sem_lx[b]).wait()
            pltpu.make_async_copy(pe_hbm.at[pl.ds(t0, CR)], bufp[b], sem_lp[b]).wait()

        def wait_store(i, b):
            r0 = base + i * CR
            pltpu.make_async_copy(bufx[b], out_hbm.at[pl.ds(r0, CR)], sem_st[b]).wait()

        for b in range(_LA):
            start_loads(b, b)

        def iteration(i, b):
            wait_loads(i, b)

            G = 16
            for r in range(CR):
                for g in range(n_col // G):
                    vs = [bufp[b][r, pl.ds((g * G + u) * L, L)]
                          for u in range(G)]
                    for u in range(G):
                        plsc.addupdate(bufx[b].at[r, pl.ds((g * G + u) * L, L)],
                                       vs[u])

            r0 = base + i * CR
            pltpu.async_copy(bufx[b], out_hbm.at[pl.ds(r0, CR)], sem_st[b])

            bn = (b + _LA) % _NB
            j = i + _LA

            def prefetch(_):
                lax.cond(i + _LA >= _NB, lambda __: wait_store(j - _NB, bn),
                         lambda __: None, 0)
                start_loads(j, bn)
                return 0

            lax.cond(j < n_chunks, prefetch, lambda _: 0, 0)

        def group(g, carry):
            for b in range(_NB):
                iteration(g * _NB + b, b)
            return carry

        lax.fori_loop(0, n_chunks // _NB, group, 0)

        for b in range(_NB):
            wait_store(n_chunks - _NB + b, (n_chunks - _NB + b) % _NB)

    return k


def _tc_add_kernel(x_ref, pe_ref, o_ref):
    o_ref[...] = x_ref[...] + pe_ref[...]


def _tc_call(x, pe_weight, b0, nb):
    B, T, D = x.shape
    BT = 2048
    grid = (T // BT, nb)
    return pl.pallas_call(
        _tc_add_kernel,
        grid=grid,
        in_specs=[
            pl.BlockSpec((1, BT, D), lambda i, b: (b + b0, i, 0)),
            pl.BlockSpec((BT, D), lambda i, b: (i, 0)),
        ],
        out_specs=pl.BlockSpec((1, BT, D), lambda i, b: (b, i, 0)),
        out_shape=jax.ShapeDtypeStruct((nb, T, D), x.dtype),
    )(x, pe_weight)


def kernel(x, pe_weight):
    B, T, D = x.shape
    sc_k = _make_sc_kernel(T, T, D)
    sc_out = sc_k(x.reshape(B * T, D), pe_weight).reshape(1, T, D)
    tc_out = _tc_call(x, pe_weight, 1, B - 1)
    return jnp.concatenate([sc_out, tc_out], axis=0)


# speedup vs baseline: 1.1948x; 1.1948x over previous
import functools

import jax
import jax.numpy as jnp
from jax import lax
from jax.experimental import pallas as pl
from jax.experimental.pallas import tpu as pltpu
from jax.experimental.pallas import tpu_sc as plsc

_NB = 4
_LA = 2


def _make_sc_kernel(R, T, D):
    info = plsc.get_sparse_core_info()
    NC, NS, L = info.num_cores, info.num_subcores, info.num_lanes
    NW = NC * NS
    rows_per_w = R // NW
    CR = 8
    n_chunks = rows_per_w // CR
    n_col = D // L

    mesh = plsc.VectorSubcoreMesh(core_axis_name="c", subcore_axis_name="s")

    scratch = (
        [pltpu.VMEM((CR, D), jnp.float32) for _ in range(_NB)]
        + [pltpu.VMEM((CR, D), jnp.float32) for _ in range(_NB)]
        + [pltpu.SemaphoreType.DMA for _ in range(3 * _NB)]
    )

    @functools.partial(
        pl.kernel,
        mesh=mesh,
        out_type=jax.ShapeDtypeStruct((R, D), jnp.float32),
        scratch_types=scratch,
    )
    def k(x_hbm, pe_hbm, out_hbm, *refs):
        bufx = refs[:_NB]
        bufp = refs[_NB:2 * _NB]
        sem_lx = refs[2 * _NB:2 * _NB + _NB]
        sem_lp = refs[2 * _NB + _NB:2 * _NB + 2 * _NB]
        sem_st = refs[2 * _NB + 2 * _NB:]

        wid = lax.axis_index("s") * NC + lax.axis_index("c")
        base = wid * rows_per_w

        def start_loads(i, b):
            r0 = base + i * CR
            t0 = lax.rem(r0, T)
            pltpu.async_copy(x_hbm.at[pl.ds(r0, CR)], bufx[b], sem_lx[b])
            pltpu.async_copy(pe_hbm.at[pl.ds(t0, CR)], bufp[b], sem_lp[b])

        def wait_loads(i, b):
            r0 = base + i * CR
            t0 = lax.rem(r0, T)
            pltpu.make_async_copy(x_hbm.at[pl.ds(r0, CR)], bufx[b], sem_lx[b]).wait()
            pltpu.make_async_copy(pe_hbm.at[pl.ds(t0, CR)], bufp[b], sem_lp[b]).wait()

        def wait_store(i, b):
            r0 = base + i * CR
            pltpu.make_async_copy(bufx[b], out_hbm.at[pl.ds(r0, CR)], sem_st[b]).wait()

        for b in range(_LA):
            start_loads(b, b)

        def iteration(i, b):
            wait_loads(i, b)

            G = 16

            @plsc.parallel_loop(0, CR)
            def _row(r):
                for g in range(n_col // G):
                    vs = [bufp[b][r, pl.ds((g * G + u) * L, L)]
                          for u in range(G)]
                    for u in range(G):
                        plsc.addupdate(bufx[b].at[r, pl.ds((g * G + u) * L, L)],
                                       vs[u])
            r0 = base + i * CR
            pltpu.async_copy(bufx[b], out_hbm.at[pl.ds(r0, CR)], sem_st[b])

            bn = (b + _LA) % _NB
            j = i + _LA

            def prefetch(_):
                lax.cond(i + _LA >= _NB, lambda __: wait_store(j - _NB, bn),
                         lambda __: None, 0)
                start_loads(j, bn)
                return 0

            lax.cond(j < n_chunks, prefetch, lambda _: 0, 0)

        def group(g, carry):
            for b in range(_NB):
                iteration(g * _NB + b, b)
            return carry

        lax.fori_loop(0, n_chunks // _NB, group, 0)

        for b in range(_NB):
            wait_store(n_chunks - _NB + b, (n_chunks - _NB + b) % _NB)

    return k


def kernel(x, pe_weight):
    B, T, D = x.shape
    R = B * T
    k = _make_sc_kernel(R, T, D)
    out = k(x.reshape(R, D), pe_weight)
    return out.reshape(B, T, D)


# speedup vs baseline: 1.4580x; 1.2204x over previous
import functools

import jax
import jax.numpy as jnp
from jax import lax
from jax.experimental import pallas as pl
from jax.experimental.pallas import tpu as pltpu
from jax.experimental.pallas import tpu_sc as plsc

_NB = 4
_LA = 2


def _make_sc_kernel(B, T, D):
    info = plsc.get_sparse_core_info()
    NC, NS, L = info.num_cores, info.num_subcores, info.num_lanes
    NW = NC * NS
    rows_per_w = T // NW
    CR = 4
    n_chunks = rows_per_w // CR
    n_col = D // L

    mesh = plsc.VectorSubcoreMesh(core_axis_name="c", subcore_axis_name="s")

    scratch = (
        [pltpu.VMEM((CR, D), jnp.float32) for _ in range(_NB * B)]
        + [pltpu.VMEM((CR, D), jnp.float32) for _ in range(_NB)]
        + [pltpu.SemaphoreType.DMA for _ in range(3 * _NB)]
    )

    @functools.partial(
        pl.kernel,
        mesh=mesh,
        out_type=jax.ShapeDtypeStruct((B * T, D), jnp.float32),
        scratch_types=scratch,
    )
    def k(x_hbm, pe_hbm, out_hbm, *refs):
        bufx = [refs[s * B:(s + 1) * B] for s in range(_NB)]
        bufp = refs[_NB * B:_NB * B + _NB]
        sem_lx = refs[_NB * B + _NB:_NB * B + 2 * _NB]
        sem_lp = refs[_NB * B + 2 * _NB:_NB * B + 3 * _NB]
        sem_st = refs[_NB * B + 3 * _NB:]

        wid = lax.axis_index("s") * NC + lax.axis_index("c")
        base = wid * rows_per_w

        def start_loads(i, s):
            t0 = base + i * CR
            for bb in range(B):
                pltpu.async_copy(x_hbm.at[pl.ds(bb * T + t0, CR)],
                                 bufx[s][bb], sem_lx[s])
            pltpu.async_copy(pe_hbm.at[pl.ds(t0, CR)], bufp[s], sem_lp[s])

        def wait_loads(i, s):
            t0 = base + i * CR
            for bb in range(B):
                pltpu.make_async_copy(x_hbm.at[pl.ds(bb * T + t0, CR)],
                                      bufx[s][bb], sem_lx[s]).wait()
            pltpu.make_async_copy(pe_hbm.at[pl.ds(t0, CR)], bufp[s],
                                  sem_lp[s]).wait()

        def wait_store(i, s):
            t0 = base + i * CR
            for bb in range(B):
                pltpu.make_async_copy(bufx[s][bb],
                                      out_hbm.at[pl.ds(bb * T + t0, CR)],
                                      sem_st[s]).wait()

        for s in range(_LA):
            start_loads(s, s)

        def iteration(i, s):
            wait_loads(i, s)

            G = 8

            @plsc.parallel_loop(0, CR)
            def _row(r):
                for g in range(n_col // G):
                    vs = [bufp[s][r, pl.ds((g * G + u) * L, L)]
                          for u in range(G)]
                    for bb in range(B):
                        for u in range(G):
                            plsc.addupdate(
                                bufx[s][bb].at[r, pl.ds((g * G + u) * L, L)],
                                vs[u])

            t0 = base + i * CR
            for bb in range(B):
                pltpu.async_copy(bufx[s][bb],
                                 out_hbm.at[pl.ds(bb * T + t0, CR)], sem_st[s])

            sn = (s + _LA) % _NB
            j = i + _LA

            def prefetch(_):
                lax.cond(i + _LA >= _NB, lambda __: wait_store(j - _NB, sn),
                         lambda __: None, 0)
                start_loads(j, sn)
                return 0

            lax.cond(j < n_chunks, prefetch, lambda _: 0, 0)

        def group(g, carry):
            for s in range(_NB):
                iteration(g * _NB + s, s)
            return carry

        lax.fori_loop(0, n_chunks // _NB, group, 0)

        for s in range(_NB):
            wait_store(n_chunks - _NB + s, (n_chunks - _NB + s) % _NB)

    return k


def kernel(x, pe_weight):
    B, T, D = x.shape
    k = _make_sc_kernel(B, T, D)
    out = k(x.reshape(B * T, D), pe_weight)
    return out.reshape(B, T, D)


# speedup vs baseline: 1.5516x; 1.0642x over previous
import functools

import jax
import jax.numpy as jnp
from jax import lax
from jax.experimental import pallas as pl
from jax.experimental.pallas import tpu as pltpu
from jax.experimental.pallas import tpu_sc as plsc

_NB = 3
_LA = 2


def _make_sc_kernel(B, T, D):
    info = plsc.get_sparse_core_info()
    NC, NS, L = info.num_cores, info.num_subcores, info.num_lanes
    NW = NC * NS
    rows_per_w = T // NW
    CR = 8
    n_chunks = rows_per_w // CR
    n_col = D // L

    mesh = plsc.VectorSubcoreMesh(core_axis_name="c", subcore_axis_name="s")

    scratch = (
        [pltpu.VMEM((CR, D), jnp.float32) for _ in range(_NB * B)]
        + [pltpu.VMEM((CR, D), jnp.float32) for _ in range(_NB)]
        + [pltpu.SemaphoreType.DMA for _ in range(3 * _NB)]
    )

    @functools.partial(
        pl.kernel,
        mesh=mesh,
        out_type=jax.ShapeDtypeStruct((B * T, D), jnp.float32),
        scratch_types=scratch,
    )
    def k(x_hbm, pe_hbm, out_hbm, *refs):
        bufx = [refs[s * B:(s + 1) * B] for s in range(_NB)]
        bufp = refs[_NB * B:_NB * B + _NB]
        sem_lx = refs[_NB * B + _NB:_NB * B + 2 * _NB]
        sem_lp = refs[_NB * B + 2 * _NB:_NB * B + 3 * _NB]
        sem_st = refs[_NB * B + 3 * _NB:]

        wid = lax.axis_index("s") * NC + lax.axis_index("c")
        base = wid * rows_per_w

        def start_loads(i, s):
            t0 = base + i * CR
            for bb in range(B):
                pltpu.async_copy(x_hbm.at[pl.ds(bb * T + t0, CR)],
                                 bufx[s][bb], sem_lx[s])
            pltpu.async_copy(pe_hbm.at[pl.ds(t0, CR)], bufp[s], sem_lp[s])

        def wait_loads(i, s):
            t0 = base + i * CR
            for bb in range(B):
                pltpu.make_async_copy(x_hbm.at[pl.ds(bb * T + t0, CR)],
                                      bufx[s][bb], sem_lx[s]).wait()
            pltpu.make_async_copy(pe_hbm.at[pl.ds(t0, CR)], bufp[s],
                                  sem_lp[s]).wait()

        def wait_store(i, s):
            t0 = base + i * CR
            for bb in range(B):
                pltpu.make_async_copy(bufx[s][bb],
                                      out_hbm.at[pl.ds(bb * T + t0, CR)],
                                      sem_st[s]).wait()

        for s in range(_LA):
            start_loads(s, s)

        def iteration(i, s):
            wait_loads(i, s)

            G = 8

            @plsc.parallel_loop(0, CR)
            def _row(r):
                for g in range(n_col // G):
                    vs = [bufp[s][r, pl.ds((g * G + u) * L, L)]
                          for u in range(G)]
                    for bb in range(B):
                        for u in range(G):
                            plsc.addupdate(
                                bufx[s][bb].at[r, pl.ds((g * G + u) * L, L)],
                                vs[u])

            t0 = base + i * CR
            for bb in range(B):
                pltpu.async_copy(bufx[s][bb],
                                 out_hbm.at[pl.ds(bb * T + t0, CR)], sem_st[s])

            sn = (s + _LA) % _NB
            j = i + _LA

            if isinstance(i, int):
                if j < n_chunks:
                    if j - _NB >= 0:
                        wait_store(j - _NB, sn)
                    start_loads(j, sn)
            else:
                def prefetch(_):
                    lax.cond(i + _LA >= _NB,
                             lambda __: wait_store(j - _NB, sn),
                             lambda __: None, 0)
                    start_loads(j, sn)
                    return 0

                lax.cond(j < n_chunks, prefetch, lambda _: 0, 0)

        def group(g, carry):
            for s in range(_NB):
                iteration(g * _NB + s, s)
            return carry

        n_groups = n_chunks // _NB
        lax.fori_loop(0, n_groups, group, 0)
        for i in range(n_groups * _NB, n_chunks):
            iteration(i, i % _NB)

        for s in range(_NB):
            wait_store(n_chunks - _NB + s, (n_chunks - _NB + s) % _NB)

    return k


def kernel(x, pe_weight):
    B, T, D = x.shape
    k = _make_sc_kernel(B, T, D)
    out = k(x.reshape(B * T, D), pe_weight)
    return out.reshape(B, T, D)
